# Initial kernel scaffold; baseline (speedup 1.0000x reference)
import jax
import jax.numpy as jnp
from jax.experimental import pallas as pl


def kernel(x, edge_index, edge_attr, batch, enc_W, enc_b, conv0_W, conv0_b, bn0_g, bn0_b, bn0_rm, bn0_rv, conv1_W, conv1_b, bn1_g, bn1_b, bn1_rm, bn1_rv, head_W1, head_b1, head_W2, head_b2, head_W3, head_b3):
    raise NotImplementedError("write your pallas kernel here")


# speedup vs baseline: 7.1696x; 7.1696x over previous
import functools

import jax
import jax.numpy as jnp
from jax import lax
from jax.experimental import pallas as pl
from jax.experimental.pallas import tpu as pltpu
from jax.experimental.pallas import tpu_sc as plsc

N = 10000
E = 320000
HID = 128
OUT = 21

NP = 10240
EROWS = 2560
CH = 128
SLAB = NP // 16
TROWS = EROWS // 32


def _sc_deg_body(dst_hbm, degp_hbm, deg_v, idx_v):
    c = lax.axis_index("c")
    s = lax.axis_index("s")
    wid = s * 2 + c

    def zbody(j, carry):
        deg_v[pl.ds(j * 16, 16)] = jnp.zeros((16,), jnp.float32)
        return carry

    lax.fori_loop(0, NP // 16, zbody, 0)

    ones = jnp.full((16,), 1.0, jnp.float32)
    base = wid * TROWS

    def body(j, carry):
        pltpu.sync_copy(dst_hbm.at[pl.ds(base + j * 4, 4)], idx_v)
        for i in range(4):
            for k in range(CH // 16):
                idx = idx_v[i, pl.ds(k * 16, 16)]
                plsc.addupdate_scatter(deg_v, [idx], ones)
        return carry

    lax.fori_loop(0, TROWS // 4, body, 0)
    pltpu.sync_copy(deg_v, degp_hbm.at[wid])


def _sc_agg_body(hs_hbm, src_hbm, dst_hbm, acc_hbm, acc_sh, sidx_v, didx_v,
                 rows_v, sem0, sem1):
    c = lax.axis_index("c")
    s = lax.axis_index("s")
    pltpu.sync_copy(hs_hbm.at[pl.ds(s * SLAB, SLAB)],
                    acc_sh.at[pl.ds(s * SLAB, SLAB)])
    plsc.subcore_barrier()

    wid = s * 2 + c
    base = wid * TROWS
    sems = (sem0, sem1)

    def body(j, carry):
        pltpu.sync_copy(src_hbm.at[pl.ds(base + j * 4, 4)], sidx_v)
        pltpu.sync_copy(dst_hbm.at[pl.ds(base + j * 4, 4)], didx_v)
        descs = [
            pltpu.async_copy(hs_hbm.at[sidx_v.at[0]], rows_v.at[0], sem0),
            pltpu.async_copy(hs_hbm.at[sidx_v.at[1]], rows_v.at[1], sem1),
        ]
        for i in range(4):
            descs[i].wait()
            pltpu.sync_copy(rows_v.at[i % 2], acc_sh.at[didx_v.at[i]],
                            add=True)
            if i + 2 < 4:
                descs.append(
                    pltpu.async_copy(hs_hbm.at[sidx_v.at[i + 2]],
                                     rows_v.at[i % 2], sems[i % 2]))
        return carry

    lax.fori_loop(0, TROWS // 4, body, 0)
    plsc.subcore_barrier()
    pltpu.sync_copy(acc_sh.at[pl.ds(s * SLAB, SLAB)],
                    acc_hbm.at[pl.ds(c * NP + s * SLAB, SLAB)])


@functools.cache
def _sc_kernels():
    mesh = plsc.VectorSubcoreMesh(core_axis_name="c", subcore_axis_name="s")
    params = pltpu.CompilerParams(needs_layout_passes=False)
    sc_deg = pl.kernel(
        _sc_deg_body,
        out_type=jax.ShapeDtypeStruct((32, NP), jnp.float32),
        mesh=mesh,
        compiler_params=params,
        scratch_types=[
            pltpu.VMEM((NP,), jnp.float32),
            pltpu.VMEM((4, CH), jnp.int32),
        ],
    )
    sc_agg = pl.kernel(
        _sc_agg_body,
        out_type=jax.ShapeDtypeStruct((2 * NP, HID), jnp.float32),
        mesh=mesh,
        compiler_params=params,
        scratch_types=[
            pltpu.VMEM_SHARED((NP, HID), jnp.float32),
            pltpu.VMEM((4, CH), jnp.int32),
            pltpu.VMEM((4, CH), jnp.int32),
            pltpu.VMEM((2, CH, HID), jnp.float32),
            pltpu.SemaphoreType.DMA,
            pltpu.SemaphoreType.DMA,
        ],
    )
    return sc_deg, sc_agg


_B = 1024


def _dinv_from(degp_ref):
    deg = 1.0 + jnp.sum(degp_ref[...], axis=0)
    return lax.rsqrt(deg)


def _tc1_body(x_ref, degp_ref, encW_ref, encb_ref, W0_ref, hs_ref):
    dinv = _dinv_from(degp_ref)
    h = jnp.dot(x_ref[...], encW_ref[...].T,
                preferred_element_type=jnp.float32) + encb_ref[...]
    hs_ref[...] = jnp.dot(h, W0_ref[...].T,
                          preferred_element_type=jnp.float32) * dinv[:, None]


def _mid_layer(acc_ref, hs_ref, degp_ref, b_ref, g_ref, bb_ref, rm_ref,
               rv_ref):
    dinv = _dinv_from(degp_ref)
    a = acc_ref[0] + acc_ref[1] - hs_ref[...]
    h = a * dinv[:, None] + b_ref[...]
    scale = g_ref[...] * lax.rsqrt(rv_ref[...] + 1e-5)
    h = (h - rm_ref[...]) * scale + bb_ref[...]
    return jnp.maximum(h, 0.0), dinv


def _tc2_body(acc_ref, hsin_ref, degp_ref, b0_ref, g_ref, bb_ref, rm_ref,
              rv_ref, W1_ref, hs_ref):
    h, dinv = _mid_layer(acc_ref, hsin_ref, degp_ref, b0_ref, g_ref, bb_ref,
                         rm_ref, rv_ref)
    hs_ref[...] = jnp.dot(h, W1_ref[...].T,
                          preferred_element_type=jnp.float32) * dinv[:, None]


def _tc3_body(acc_ref, hsin_ref, degp_ref, b1_ref, g_ref, bb_ref, rm_ref,
              rv_ref, hW1_ref, hb1_ref, hW2_ref, hb2_ref, hW3_ref, hb3_ref,
              y_ref):
    h, _ = _mid_layer(acc_ref, hsin_ref, degp_ref, b1_ref, g_ref, bb_ref,
                      rm_ref, rv_ref)
    h = jnp.maximum(jnp.dot(h, hW1_ref[...].T,
                            preferred_element_type=jnp.float32)
                    + hb1_ref[...], 0.0)
    h = jnp.maximum(jnp.dot(h, hW2_ref[...].T,
                            preferred_element_type=jnp.float32)
                    + hb2_ref[...], 0.0)
    y_ref[...] = jnp.dot(h, hW3_ref[...].T,
                         preferred_element_type=jnp.float32) + hb3_ref[...]


def _full(shape):
    nd = len(shape)
    return pl.BlockSpec(shape, lambda i, _n=nd: (0,) * _n)


def _rows(block):
    return pl.BlockSpec(block, lambda i: (i,) + (0,) * (len(block) - 1))


def _degp_spec():
    return pl.BlockSpec((32, _B), lambda i: (0, i))


def _acc_spec():
    return pl.BlockSpec((2, _B, HID), lambda i: (0, i, 0))


def _tc1(x_p, degp, enc_W, enc_b, conv0_W):
    return pl.pallas_call(
        _tc1_body,
        grid=(NP // _B,),
        in_specs=[_rows((_B, 14)), _degp_spec(), _full((HID, 14)),
                  _full((HID,)), _full((HID, HID))],
        out_specs=_rows((_B, HID)),
        out_shape=jax.ShapeDtypeStruct((NP, HID), jnp.float32),
    )(x_p, degp, enc_W, enc_b, conv0_W)


def _tc2(acc, hs, degp, b0, g, bb, rm, rv, W1):
    return pl.pallas_call(
        _tc2_body,
        grid=(NP // _B,),
        in_specs=[_acc_spec(), _rows((_B, HID)), _degp_spec()]
                 + [_full((HID,))] * 5 + [_full((HID, HID))],
        out_specs=_rows((_B, HID)),
        out_shape=jax.ShapeDtypeStruct((NP, HID), jnp.float32),
    )(acc, hs, degp, b0, g, bb, rm, rv, W1)


def _tc3(acc, hs, degp, b1, g, bb, rm, rv, hW1, hb1, hW2, hb2, hW3p, hb3p):
    return pl.pallas_call(
        _tc3_body,
        grid=(NP // _B,),
        in_specs=[_acc_spec(), _rows((_B, HID)), _degp_spec()]
                 + [_full((HID,))] * 5
                 + [_full((HID, HID)), _full((HID,)), _full((HID, HID)),
                    _full((HID,)), _full((32, HID)), _full((32,))],
        out_specs=_rows((_B, 32)),
        out_shape=jax.ShapeDtypeStruct((NP, 32), jnp.float32),
    )(acc, hs, degp, b1, g, bb, rm, rv, hW1, hb1, hW2, hb2, hW3p, hb3p)


def kernel(x, edge_index, edge_attr, batch, enc_W, enc_b, conv0_W, conv0_b,
           bn0_g, bn0_b, bn0_rm, bn0_rv, conv1_W, conv1_b, bn1_g, bn1_b,
           bn1_rm, bn1_rv, head_W1, head_b1, head_W2, head_b2, head_W3,
           head_b3):
    del edge_attr, batch
    PAD_IDX = N + 100

    src2 = jnp.concatenate(
        [edge_index[0],
         jnp.full((EROWS * CH - E,), PAD_IDX, jnp.int32)]).reshape(EROWS, CH)
    dst2 = jnp.concatenate(
        [edge_index[1],
         jnp.full((EROWS * CH - E,), PAD_IDX, jnp.int32)]).reshape(EROWS, CH)

    x_p = jnp.concatenate([x, jnp.zeros((NP - N, 14), jnp.float32)])

    _sc_deg, _sc_agg = _sc_kernels()
    degp = _sc_deg(dst2)

    hs0 = _tc1(x_p, degp, enc_W, enc_b, conv0_W)
    acc0 = _sc_agg(hs0, src2, dst2).reshape(2, NP, HID)

    hs1 = _tc2(acc0, hs0, degp, conv0_b, bn0_g, bn0_b, bn0_rm, bn0_rv,
               conv1_W)
    acc1 = _sc_agg(hs1, src2, dst2).reshape(2, NP, HID)

    hW3p = jnp.concatenate([head_W3, jnp.zeros((32 - OUT, HID), jnp.float32)])
    hb3p = jnp.concatenate([head_b3, jnp.zeros((32 - OUT,), jnp.float32)])
    y = _tc3(acc1, hs1, degp, conv1_b, bn1_g, bn1_b, bn1_rm, bn1_rv,
             head_W1, head_b1, head_W2, head_b2, hW3p, hb3p)
    return y[:N, :OUT]


# speedup vs baseline: 15.7284x; 2.1938x over previous
import functools

import jax
import jax.numpy as jnp
from jax import lax
from jax.experimental import pallas as pl
from jax.experimental.pallas import tpu as pltpu
from jax.experimental.pallas import tpu_sc as plsc

N = 10000
E = 320000
HID = 128
OUT = 21

NP = 10240
EROWS = 2560
CH = 128
SLAB = NP // 16
TROWS = EROWS // 32


def _sc_deg_body(dst_hbm, degp_hbm, deg_v, idx_v):
    c = lax.axis_index("c")
    s = lax.axis_index("s")
    wid = s * 2 + c

    def zbody(j, carry):
        deg_v[pl.ds(j * 16, 16)] = jnp.zeros((16,), jnp.float32)
        return carry

    lax.fori_loop(0, NP // 16, zbody, 0)

    ones = jnp.full((16,), 1.0, jnp.float32)
    base = wid * TROWS

    def body(j, carry):
        pltpu.sync_copy(dst_hbm.at[pl.ds(base + j * 4, 4)], idx_v)
        for i in range(4):
            for k in range(CH // 16):
                idx = idx_v[i, pl.ds(k * 16, 16)]
                plsc.addupdate_scatter(deg_v, [idx], ones)
        return carry

    lax.fori_loop(0, TROWS // 4, body, 0)
    pltpu.sync_copy(deg_v, degp_hbm.at[wid])


F = 64
SROWS = EROWS // 16


def _sc_agg_body(hs_hbm, src_hbm, dst_hbm, acc_hbm, hs_sh, acc_sh, sidx_v,
                 didx_v, rows_v, sem0, sem1):
    c = lax.axis_index("c")
    s = lax.axis_index("s")
    pltpu.sync_copy(hs_hbm.at[pl.ds(c * NP + s * SLAB, SLAB)],
                    hs_sh.at[pl.ds(s * SLAB, SLAB)])
    pltpu.sync_copy(hs_hbm.at[pl.ds(c * NP + s * SLAB, SLAB)],
                    acc_sh.at[pl.ds(s * SLAB, SLAB)])
    plsc.subcore_barrier()

    base = s * SROWS
    sems = (sem0, sem1)

    def body(j, carry):
        pltpu.sync_copy(src_hbm.at[pl.ds(base + j * 4, 4)], sidx_v)
        pltpu.sync_copy(dst_hbm.at[pl.ds(base + j * 4, 4)], didx_v)
        descs = [
            pltpu.async_copy(hs_sh.at[sidx_v.at[0]], rows_v.at[0], sem0),
            pltpu.async_copy(hs_sh.at[sidx_v.at[1]], rows_v.at[1], sem1),
        ]
        for i in range(4):
            descs[i].wait()
            pltpu.sync_copy(rows_v.at[i % 2], acc_sh.at[didx_v.at[i]],
                            add=True)
            if i + 2 < 4:
                descs.append(
                    pltpu.async_copy(hs_sh.at[sidx_v.at[i + 2]],
                                     rows_v.at[i % 2], sems[i % 2]))
        return carry

    lax.fori_loop(0, SROWS // 4, body, 0)
    plsc.subcore_barrier()
    pltpu.sync_copy(acc_sh.at[pl.ds(s * SLAB, SLAB)],
                    acc_hbm.at[pl.ds(c * NP + s * SLAB, SLAB)])


@functools.cache
def _sc_kernels():
    mesh = plsc.VectorSubcoreMesh(core_axis_name="c", subcore_axis_name="s")
    params = pltpu.CompilerParams(needs_layout_passes=False,
                                  use_tc_tiling_on_sc=False)
    sc_deg = pl.kernel(
        _sc_deg_body,
        out_type=jax.ShapeDtypeStruct((32, NP), jnp.float32),
        mesh=mesh,
        compiler_params=params,
        scratch_types=[
            pltpu.VMEM((NP,), jnp.float32),
            pltpu.VMEM((4, CH), jnp.int32),
        ],
    )
    sc_agg = pl.kernel(
        _sc_agg_body,
        out_type=jax.ShapeDtypeStruct((2 * NP, F), jnp.float32),
        mesh=mesh,
        compiler_params=params,
        scratch_types=[
            pltpu.VMEM_SHARED((NP, F), jnp.float32),
            pltpu.VMEM_SHARED((NP, F), jnp.float32),
            pltpu.VMEM((4, CH), jnp.int32),
            pltpu.VMEM((4, CH), jnp.int32),
            pltpu.VMEM((2, CH, F), jnp.float32),
            pltpu.SemaphoreType.DMA,
            pltpu.SemaphoreType.DMA,
        ],
    )
    return sc_deg, sc_agg


_B = 1024


def _dinv_from(degp_ref):
    deg = 1.0 + jnp.sum(degp_ref[...], axis=0)
    return lax.rsqrt(deg)


def _tc1_body(x_ref, degp_ref, encW_ref, encb_ref, W0_ref, hs_ref):
    dinv = _dinv_from(degp_ref)
    h = jnp.dot(x_ref[...], encW_ref[...].T,
                preferred_element_type=jnp.float32) + encb_ref[...]
    hs = jnp.dot(h, W0_ref[...].T,
                 preferred_element_type=jnp.float32) * dinv[:, None]
    hs_ref[0] = hs[:, :F]
    hs_ref[1] = hs[:, F:]


def _mid_layer(acc_ref, degp_ref, b_ref, g_ref, bb_ref, rm_ref, rv_ref):
    dinv = _dinv_from(degp_ref)
    a = jnp.concatenate([acc_ref[0], acc_ref[1]], axis=1)
    h = a * dinv[:, None] + b_ref[...]
    scale = g_ref[...] * lax.rsqrt(rv_ref[...] + 1e-5)
    h = (h - rm_ref[...]) * scale + bb_ref[...]
    return jnp.maximum(h, 0.0), dinv


def _tc2_body(acc_ref, degp_ref, b0_ref, g_ref, bb_ref, rm_ref,
              rv_ref, W1_ref, hs_ref):
    h, dinv = _mid_layer(acc_ref, degp_ref, b0_ref, g_ref, bb_ref,
                         rm_ref, rv_ref)
    hs = jnp.dot(h, W1_ref[...].T,
                 preferred_element_type=jnp.float32) * dinv[:, None]
    hs_ref[0] = hs[:, :F]
    hs_ref[1] = hs[:, F:]


def _tc3_body(acc_ref, degp_ref, b1_ref, g_ref, bb_ref, rm_ref,
              rv_ref, hW1_ref, hb1_ref, hW2_ref, hb2_ref, hW3_ref, hb3_ref,
              y_ref):
    h, _ = _mid_layer(acc_ref, degp_ref, b1_ref, g_ref, bb_ref,
                      rm_ref, rv_ref)
    h = jnp.maximum(jnp.dot(h, hW1_ref[...].T,
                            preferred_element_type=jnp.float32)
                    + hb1_ref[...], 0.0)
    h = jnp.maximum(jnp.dot(h, hW2_ref[...].T,
                            preferred_element_type=jnp.float32)
                    + hb2_ref[...], 0.0)
    y_ref[...] = jnp.dot(h, hW3_ref[...].T,
                         preferred_element_type=jnp.float32) + hb3_ref[...]


def _full(shape):
    nd = len(shape)
    return pl.BlockSpec(shape, lambda i, _n=nd: (0,) * _n)


def _rows(block):
    return pl.BlockSpec(block, lambda i: (i,) + (0,) * (len(block) - 1))


def _degp_spec():
    return pl.BlockSpec((32, _B), lambda i: (0, i))


def _acc_spec():
    return pl.BlockSpec((2, _B, F), lambda i: (0, i, 0))


def _tc1(x_p, degp, enc_W, enc_b, conv0_W):
    return pl.pallas_call(
        _tc1_body,
        grid=(NP // _B,),
        in_specs=[_rows((_B, 14)), _degp_spec(), _full((HID, 14)),
                  _full((HID,)), _full((HID, HID))],
        out_specs=_acc_spec(),
        out_shape=jax.ShapeDtypeStruct((2, NP, F), jnp.float32),
    )(x_p, degp, enc_W, enc_b, conv0_W)


def _tc2(acc, degp, b0, g, bb, rm, rv, W1):
    return pl.pallas_call(
        _tc2_body,
        grid=(NP // _B,),
        in_specs=[_acc_spec(), _degp_spec()]
                 + [_full((HID,))] * 5 + [_full((HID, HID))],
        out_specs=_acc_spec(),
        out_shape=jax.ShapeDtypeStruct((2, NP, F), jnp.float32),
    )(acc, degp, b0, g, bb, rm, rv, W1)


def _tc3(acc, degp, b1, g, bb, rm, rv, hW1, hb1, hW2, hb2, hW3p, hb3p):
    return pl.pallas_call(
        _tc3_body,
        grid=(NP // _B,),
        in_specs=[_acc_spec(), _degp_spec()]
                 + [_full((HID,))] * 5
                 + [_full((HID, HID)), _full((HID,)), _full((HID, HID)),
                    _full((HID,)), _full((32, HID)), _full((32,))],
        out_specs=_rows((_B, 32)),
        out_shape=jax.ShapeDtypeStruct((NP, 32), jnp.float32),
    )(acc, degp, b1, g, bb, rm, rv, hW1, hb1, hW2, hb2, hW3p, hb3p)


def kernel(x, edge_index, edge_attr, batch, enc_W, enc_b, conv0_W, conv0_b,
           bn0_g, bn0_b, bn0_rm, bn0_rv, conv1_W, conv1_b, bn1_g, bn1_b,
           bn1_rm, bn1_rv, head_W1, head_b1, head_W2, head_b2, head_W3,
           head_b3):
    del edge_attr, batch
    PAD_IDX = N + 100

    src2 = jnp.concatenate(
        [edge_index[0],
         jnp.full((EROWS * CH - E,), PAD_IDX, jnp.int32)]).reshape(EROWS, CH)
    dst2 = jnp.concatenate(
        [edge_index[1],
         jnp.full((EROWS * CH - E,), PAD_IDX, jnp.int32)]).reshape(EROWS, CH)

    x_p = jnp.concatenate([x, jnp.zeros((NP - N, 14), jnp.float32)])

    _sc_deg, _sc_agg = _sc_kernels()
    degp = _sc_deg(dst2)

    hs0 = _tc1(x_p, degp, enc_W, enc_b, conv0_W)
    acc0 = _sc_agg(hs0.reshape(2 * NP, F), src2, dst2).reshape(2, NP, F)

    hs1 = _tc2(acc0, degp, conv0_b, bn0_g, bn0_b, bn0_rm, bn0_rv, conv1_W)
    acc1 = _sc_agg(hs1.reshape(2 * NP, F), src2, dst2).reshape(2, NP, F)

    hW3p = jnp.concatenate([head_W3, jnp.zeros((32 - OUT, HID), jnp.float32)])
    hb3p = jnp.concatenate([head_b3, jnp.zeros((32 - OUT,), jnp.float32)])
    y = _tc3(acc1, degp, conv1_b, bn1_g, bn1_b, bn1_rm, bn1_rv,
             head_W1, head_b1, head_W2, head_b2, hW3p, hb3p)
    return y[:N, :OUT]


# speedup vs baseline: 18.0192x; 1.1456x over previous
import functools

import jax
import jax.numpy as jnp
from jax import lax
from jax.experimental import pallas as pl
from jax.experimental.pallas import tpu as pltpu
from jax.experimental.pallas import tpu_sc as plsc

N = 10000
E = 320000
HID = 128
OUT = 21

NP = 10240
EROWS = 2560
CH = 128
SLAB = NP // 16
TROWS = EROWS // 32


def _sc_deg_body(dst_hbm, degp_hbm, deg_v, idx_v):
    c = lax.axis_index("c")
    s = lax.axis_index("s")
    wid = s * 2 + c

    def zbody(j, carry):
        deg_v[pl.ds(j * 16, 16)] = jnp.zeros((16,), jnp.float32)
        return carry

    lax.fori_loop(0, NP // 16, zbody, 0)

    ones = jnp.full((16,), 1.0, jnp.float32)
    base = wid * TROWS

    def body(j, carry):
        pltpu.sync_copy(dst_hbm.at[pl.ds(base + j * 4, 4)], idx_v)
        for i in range(4):
            for k in range(CH // 16):
                idx = idx_v[i, pl.ds(k * 16, 16)]
                plsc.addupdate_scatter(deg_v, [idx], ones)
        return carry

    lax.fori_loop(0, TROWS // 4, body, 0)
    pltpu.sync_copy(deg_v, degp_hbm.at[wid])


F = 64
SROWS = EROWS // 16
NBLK = SROWS // 4


def _sc_agg_body(hs_hbm, sdx_hbm, acc_hbm, hs_sh, acc_sh, idxA, idxB, rows_v,
                 g0, g1, g2, g3, s0, s1, s2, s3, isem):
    c = lax.axis_index("c")
    s = lax.axis_index("s")
    pltpu.sync_copy(hs_hbm.at[pl.ds(c * NP + s * SLAB, SLAB)],
                    hs_sh.at[pl.ds(s * SLAB, SLAB)])
    pltpu.sync_copy(hs_hbm.at[pl.ds(c * NP + s * SLAB, SLAB)],
                    acc_sh.at[pl.ds(s * SLAB, SLAB)])
    plsc.subcore_barrier()

    gsem = (g0, g1, g2, g3)
    ssem = (s0, s1, s2, s3)
    base = s * NBLK

    def emit_block(g, bufX, bufY, nxt):
        if nxt:
            pidesc = pltpu.async_copy(sdx_hbm.at[pl.ds((base + g + 1) * 8, 8)],
                                      bufY, isem)
        sdescs = []
        for i in range(4):
            pltpu.make_async_copy(hs_sh.at[bufX.at[i]], rows_v.at[i],
                                  gsem[i]).wait()
            sdescs.append(
                pltpu.async_copy(rows_v.at[i], acc_sh.at[bufX.at[4 + i]],
                                 ssem[i], add=True))
        if nxt:
            pidesc.wait()
            for i in range(4):
                sdescs[i].wait()
                pltpu.async_copy(hs_sh.at[bufY.at[i]], rows_v.at[i], gsem[i])
        else:
            for i in range(4):
                sdescs[i].wait()

    pltpu.sync_copy(sdx_hbm.at[pl.ds(base * 8, 8)], idxA)
    for i in range(4):
        pltpu.async_copy(hs_sh.at[idxA.at[i]], rows_v.at[i], gsem[i])

    def body(t, carry):
        emit_block(2 * t, idxA, idxB, True)
        emit_block(2 * t + 1, idxB, idxA, True)
        return carry

    lax.fori_loop(0, (NBLK - 2) // 2, body, 0)
    emit_block(NBLK - 2, idxA, idxB, True)
    emit_block(NBLK - 1, idxB, idxA, False)

    plsc.subcore_barrier()
    pltpu.sync_copy(acc_sh.at[pl.ds(s * SLAB, SLAB)],
                    acc_hbm.at[pl.ds(c * NP + s * SLAB, SLAB)])


@functools.cache
def _sc_kernels():
    mesh = plsc.VectorSubcoreMesh(core_axis_name="c", subcore_axis_name="s")
    params = pltpu.CompilerParams(needs_layout_passes=False,
                                  use_tc_tiling_on_sc=False)
    sc_deg = pl.kernel(
        _sc_deg_body,
        out_type=jax.ShapeDtypeStruct((32, NP), jnp.float32),
        mesh=mesh,
        compiler_params=params,
        scratch_types=[
            pltpu.VMEM((NP,), jnp.float32),
            pltpu.VMEM((4, CH), jnp.int32),
        ],
    )
    sc_agg = pl.kernel(
        _sc_agg_body,
        out_type=jax.ShapeDtypeStruct((2 * NP, F), jnp.float32),
        mesh=mesh,
        compiler_params=params,
        scratch_types=[
            pltpu.VMEM_SHARED((NP, F), jnp.float32),
            pltpu.VMEM_SHARED((NP, F), jnp.float32),
            pltpu.VMEM((8, CH), jnp.int32),
            pltpu.VMEM((8, CH), jnp.int32),
            pltpu.VMEM((4, CH, F), jnp.float32),
        ] + [pltpu.SemaphoreType.DMA] * 9,
    )
    return sc_deg, sc_agg


_B = 1024


def _dinv_from(degp_ref):
    deg = 1.0 + jnp.sum(degp_ref[...], axis=0)
    return lax.rsqrt(deg)


def _tc1_body(x_ref, degp_ref, encW_ref, encb_ref, W0_ref, hs_ref):
    dinv = _dinv_from(degp_ref)
    h = jnp.dot(x_ref[...], encW_ref[...].T,
                preferred_element_type=jnp.float32) + encb_ref[...]
    hs = jnp.dot(h, W0_ref[...].T,
                 preferred_element_type=jnp.float32) * dinv[:, None]
    hs_ref[0] = hs[:, :F]
    hs_ref[1] = hs[:, F:]


def _mid_layer(acc_ref, degp_ref, b_ref, g_ref, bb_ref, rm_ref, rv_ref):
    dinv = _dinv_from(degp_ref)
    a = jnp.concatenate([acc_ref[0], acc_ref[1]], axis=1)
    h = a * dinv[:, None] + b_ref[...]
    scale = g_ref[...] * lax.rsqrt(rv_ref[...] + 1e-5)
    h = (h - rm_ref[...]) * scale + bb_ref[...]
    return jnp.maximum(h, 0.0), dinv


def _tc2_body(acc_ref, degp_ref, b0_ref, g_ref, bb_ref, rm_ref,
              rv_ref, W1_ref, hs_ref):
    h, dinv = _mid_layer(acc_ref, degp_ref, b0_ref, g_ref, bb_ref,
                         rm_ref, rv_ref)
    hs = jnp.dot(h, W1_ref[...].T,
                 preferred_element_type=jnp.float32) * dinv[:, None]
    hs_ref[0] = hs[:, :F]
    hs_ref[1] = hs[:, F:]


def _tc3_body(acc_ref, degp_ref, b1_ref, g_ref, bb_ref, rm_ref,
              rv_ref, hW1_ref, hb1_ref, hW2_ref, hb2_ref, hW3_ref, hb3_ref,
              y_ref):
    h, _ = _mid_layer(acc_ref, degp_ref, b1_ref, g_ref, bb_ref,
                      rm_ref, rv_ref)
    h = jnp.maximum(jnp.dot(h, hW1_ref[...].T,
                            preferred_element_type=jnp.float32)
                    + hb1_ref[...], 0.0)
    h = jnp.maximum(jnp.dot(h, hW2_ref[...].T,
                            preferred_element_type=jnp.float32)
                    + hb2_ref[...], 0.0)
    y_ref[...] = jnp.dot(h, hW3_ref[...].T,
                         preferred_element_type=jnp.float32) + hb3_ref[...]


def _full(shape):
    nd = len(shape)
    return pl.BlockSpec(shape, lambda i, _n=nd: (0,) * _n)


def _rows(block):
    return pl.BlockSpec(block, lambda i: (i,) + (0,) * (len(block) - 1))


def _degp_spec():
    return pl.BlockSpec((32, _B), lambda i: (0, i))


def _acc_spec():
    return pl.BlockSpec((2, _B, F), lambda i: (0, i, 0))


def _tc1(x_p, degp, enc_W, enc_b, conv0_W):
    return pl.pallas_call(
        _tc1_body,
        grid=(NP // _B,),
        in_specs=[_rows((_B, 14)), _degp_spec(), _full((HID, 14)),
                  _full((HID,)), _full((HID, HID))],
        out_specs=_acc_spec(),
        out_shape=jax.ShapeDtypeStruct((2, NP, F), jnp.float32),
    )(x_p, degp, enc_W, enc_b, conv0_W)


def _tc2(acc, degp, b0, g, bb, rm, rv, W1):
    return pl.pallas_call(
        _tc2_body,
        grid=(NP // _B,),
        in_specs=[_acc_spec(), _degp_spec()]
                 + [_full((HID,))] * 5 + [_full((HID, HID))],
        out_specs=_acc_spec(),
        out_shape=jax.ShapeDtypeStruct((2, NP, F), jnp.float32),
    )(acc, degp, b0, g, bb, rm, rv, W1)


def _tc3(acc, degp, b1, g, bb, rm, rv, hW1, hb1, hW2, hb2, hW3p, hb3p):
    return pl.pallas_call(
        _tc3_body,
        grid=(NP // _B,),
        in_specs=[_acc_spec(), _degp_spec()]
                 + [_full((HID,))] * 5
                 + [_full((HID, HID)), _full((HID,)), _full((HID, HID)),
                    _full((HID,)), _full((32, HID)), _full((32,))],
        out_specs=_rows((_B, 32)),
        out_shape=jax.ShapeDtypeStruct((NP, 32), jnp.float32),
    )(acc, degp, b1, g, bb, rm, rv, hW1, hb1, hW2, hb2, hW3p, hb3p)


def kernel(x, edge_index, edge_attr, batch, enc_W, enc_b, conv0_W, conv0_b,
           bn0_g, bn0_b, bn0_rm, bn0_rv, conv1_W, conv1_b, bn1_g, bn1_b,
           bn1_rm, bn1_rv, head_W1, head_b1, head_W2, head_b2, head_W3,
           head_b3):
    del edge_attr, batch
    PAD_IDX = N + 100

    src2 = jnp.concatenate(
        [edge_index[0],
         jnp.full((EROWS * CH - E,), PAD_IDX, jnp.int32)]).reshape(EROWS, CH)
    dst2 = jnp.concatenate(
        [edge_index[1],
         jnp.full((EROWS * CH - E,), PAD_IDX, jnp.int32)]).reshape(EROWS, CH)

    x_p = jnp.concatenate([x, jnp.zeros((NP - N, 14), jnp.float32)])

    sdx = jnp.concatenate([src2.reshape(EROWS // 4, 4, CH),
                           dst2.reshape(EROWS // 4, 4, CH)],
                          axis=1).reshape(2 * EROWS, CH)

    _sc_deg, _sc_agg = _sc_kernels()
    degp = _sc_deg(dst2)

    hs0 = _tc1(x_p, degp, enc_W, enc_b, conv0_W)
    acc0 = _sc_agg(hs0.reshape(2 * NP, F), sdx).reshape(2, NP, F)

    hs1 = _tc2(acc0, degp, conv0_b, bn0_g, bn0_b, bn0_rm, bn0_rv, conv1_W)
    acc1 = _sc_agg(hs1.reshape(2 * NP, F), sdx).reshape(2, NP, F)

    hW3p = jnp.concatenate([head_W3, jnp.zeros((32 - OUT, HID), jnp.float32)])
    hb3p = jnp.concatenate([head_b3, jnp.zeros((32 - OUT,), jnp.float32)])
    y = _tc3(acc1, degp, conv1_b, bn1_g, bn1_b, bn1_rm, bn1_rv,
             head_W1, head_b1, head_W2, head_b2, hW3p, hb3p)
    return y[:N, :OUT]
